# baseline (device time: 3389983 ns/iter reference)
import jax
import jax.numpy as jnp
from jax import lax
from jax.experimental import pallas as pl
from jax.experimental.pallas import tpu as pltpu

N_DEV = 4


def _allreduce_body(p_ref, out_ref, recv_ref, acc_a, acc_b,
                    local_sems, send_sems, recv_sems):
    my = lax.axis_index("i")
    left = (my - 1) % N_DEV
    right = (my + 1) % N_DEV

    m, n = out_ref.shape
    blk = m // N_DEV
    sub = 512
    n_sub = blk // sub

    barrier_sem = pltpu.get_barrier_semaphore()
    for nbr in (left, right):
        pl.semaphore_signal(barrier_sem, inc=1, device_id=(nbr,),
                            device_id_type=pl.DeviceIdType.MESH)
    pl.semaphore_wait(barrier_sem, 2)

    cp = pltpu.make_async_copy(p_ref, out_ref, local_sems.at[0])
    cp.start()
    cp.wait()

    for s in range(N_DEV - 1):
        b_send = (my - s) % N_DEV
        rdma = pltpu.make_async_remote_copy(
            src_ref=out_ref.at[pl.ds(b_send * blk, blk), :],
            dst_ref=recv_ref.at[s],
            send_sem=send_sems.at[s],
            recv_sem=recv_sems.at[s],
            device_id=(right,),
            device_id_type=pl.DeviceIdType.MESH,
        )
        rdma.start()
        rdma.wait()

        b_acc = (my - s - 1) % N_DEV
        for j in range(n_sub):
            row0 = b_acc * blk + j * sub
            c1 = pltpu.make_async_copy(
                out_ref.at[pl.ds(row0, sub), :], acc_a, local_sems.at[0])
            c2 = pltpu.make_async_copy(
                recv_ref.at[s, pl.ds(j * sub, sub), :], acc_b,
                local_sems.at[1])
            c1.start()
            c2.start()
            c1.wait()
            c2.wait()
            acc_a[...] = (acc_a[...].astype(jnp.float32)
                          + acc_b[...].astype(jnp.float32)
                          ).astype(jnp.bfloat16)
            c3 = pltpu.make_async_copy(
                acc_a, out_ref.at[pl.ds(row0, sub), :], local_sems.at[0])
            c3.start()
            c3.wait()

    for t in range(N_DEV - 1):
        b_send = (my + 1 - t) % N_DEV
        rdma = pltpu.make_async_remote_copy(
            src_ref=out_ref.at[pl.ds(b_send * blk, blk), :],
            dst_ref=out_ref.at[pl.ds(b_send * blk, blk), :],
            send_sem=send_sems.at[N_DEV - 1 + t],
            recv_sem=recv_sems.at[N_DEV - 1 + t],
            device_id=(right,),
            device_id_type=pl.DeviceIdType.MESH,
        )
        rdma.start()
        rdma.wait()


def _pallas_allreduce(partial):
    m, n = partial.shape
    blk = m // N_DEV
    out, _ = pl.pallas_call(
        _allreduce_body,
        out_shape=(
            jax.ShapeDtypeStruct((m, n), jnp.bfloat16),
            jax.ShapeDtypeStruct((N_DEV - 1, blk, n), jnp.bfloat16),
        ),
        in_specs=[pl.BlockSpec(memory_space=pltpu.MemorySpace.HBM)],
        out_specs=(
            pl.BlockSpec(memory_space=pltpu.MemorySpace.HBM),
            pl.BlockSpec(memory_space=pltpu.MemorySpace.HBM),
        ),
        scratch_shapes=[
            pltpu.VMEM((512, n), jnp.bfloat16),
            pltpu.VMEM((512, n), jnp.bfloat16),
            pltpu.SemaphoreType.DMA((2,)),
            pltpu.SemaphoreType.DMA((2 * (N_DEV - 1),)),
            pltpu.SemaphoreType.DMA((2 * (N_DEV - 1),)),
        ],
        compiler_params=pltpu.CompilerParams(collective_id=0),
    )(partial)
    return out


def kernel(x, w_mat):
    partial = jnp.dot(
        x.astype(jnp.bfloat16), w_mat.astype(jnp.bfloat16),
        preferred_element_type=jnp.float32,
    ).astype(jnp.bfloat16)

    y = _pallas_allreduce(partial).astype(jnp.float32)

    amax = jnp.max(jnp.abs(y))
    scale = amax / 127.0
    q = jnp.clip(jnp.round(y / scale), -127.0, 127.0)
    return q * scale
